# baseline (speedup 1.0000x reference)
import functools

import jax
import jax.numpy as jnp
import numpy as np
from jax import lax
from jax.experimental import pallas as pl
from jax.experimental.pallas import tpu as pltpu
from jax.experimental.pallas import tpu_sc as plsc

_SC_PARAMS = pltpu.CompilerParams(use_tc_tiling_on_sc=False)

NC = 2
NS = 16
NW = NC * NS
L = 16
CHUNK = 128
NBUF = 8


def _load_idx(e2d_hbm, pad_hbm, wid, idx_v, n_chunks, real_rows):
  wb = real_rows // n_chunks
  rem = real_rows % n_chunks

  @pl.when(wid < wb)
  def _():
    pltpu.sync_copy(e2d_hbm.at[pl.ds(wid * n_chunks, n_chunks)], idx_v)

  if rem:
    @pl.when(wid == wb)
    def _():
      pltpu.sync_copy(e2d_hbm.at[pl.ds(wb * n_chunks, rem)],
                      idx_v.at[pl.ds(0, rem)])
      pltpu.sync_copy(pad_hbm,
                      idx_v.at[pl.ds(rem, n_chunks - rem)])


def _sc_hist(dst2d, pad_dst, zeros_np, ones_chunk, n_pad, n_chunks,
             real_rows):
  mesh = plsc.VectorSubcoreMesh(core_axis_name="c", subcore_axis_name="s")
  rpt = n_pad // NS

  @functools.partial(
      pl.kernel,
      out_type=jax.ShapeDtypeStruct((NC, n_pad, L), jnp.float32),
      mesh=mesh,
      compiler_params=_SC_PARAMS,
      scratch_types=[
          pltpu.VMEM((n_chunks, CHUNK), jnp.int32),
          pltpu.VMEM((CHUNK, L), jnp.float32),
          pltpu.VMEM_SHARED((n_pad, L), jnp.float32),
          pltpu.SemaphoreType.DMA,
      ],
  )
  def hist(dst_hbm, pad_hbm, zeros_hbm, ones_hbm, out_hbm,
           dst_v, ones_v, acc, sem):
    c = lax.axis_index("c")
    s = lax.axis_index("s")
    wid = c * NS + s
    rows = pl.ds(s * rpt, rpt)
    _load_idx(dst_hbm, pad_hbm, wid, dst_v, n_chunks, real_rows)
    pltpu.sync_copy(ones_hbm, ones_v)
    pltpu.sync_copy(zeros_hbm.at[rows], acc.at[rows])
    plsc.subcore_barrier()

    @pl.loop(0, n_chunks)
    def _(j):
      pltpu.async_copy(ones_v, acc.at[dst_v.at[j]], sem, add=True)

    @pl.loop(0, n_chunks)
    def _(j):
      pltpu.make_async_copy(ones_v, acc.at[dst_v.at[j]], sem).wait()

    plsc.subcore_barrier()
    pltpu.sync_copy(acc.at[rows], out_hbm.at[c, rows])

  return hist(dst2d, pad_dst, zeros_np, ones_chunk)


def _sc_edge_pass(table, src2d, dst2d, pad_src, pad_dst, zeros_np, n_pad,
                  n_chunks, real_rows):
  mesh = plsc.VectorSubcoreMesh(core_axis_name="c", subcore_axis_name="s")
  rpt = n_pad // NS
  outer = n_chunks // NBUF

  @functools.partial(
      pl.kernel,
      out_type=jax.ShapeDtypeStruct((NC, n_pad, L), jnp.float32),
      mesh=mesh,
      compiler_params=_SC_PARAMS,
      scratch_types=[
          pltpu.VMEM((n_chunks, CHUNK), jnp.int32),
          pltpu.VMEM((n_chunks, CHUNK), jnp.int32),
          pltpu.VMEM((NBUF, CHUNK, L), jnp.float32),
          pltpu.VMEM_SHARED((n_pad, L), jnp.float32),
          pltpu.SemaphoreType.DMA((NBUF,)),
          pltpu.SemaphoreType.DMA((NBUF,)),
      ],
  )
  def edge_pass(tab_hbm, src_hbm, dst_hbm, psrc_hbm, pdst_hbm, zeros_hbm,
                out_hbm, src_v, dst_v, bufs, acc, gsem, ssem):
    c = lax.axis_index("c")
    s = lax.axis_index("s")
    wid = c * NS + s
    rows = pl.ds(s * rpt, rpt)
    _load_idx(src_hbm, psrc_hbm, wid, src_v, n_chunks, real_rows)
    _load_idx(dst_hbm, pdst_hbm, wid, dst_v, n_chunks, real_rows)
    pltpu.sync_copy(zeros_hbm.at[rows], acc.at[rows])
    plsc.subcore_barrier()

    for b in range(NBUF):
      pltpu.async_copy(tab_hbm.at[src_v.at[b]], bufs.at[b], gsem.at[b])

    @pl.loop(0, outer)
    def _(t):
      base = t * NBUF
      for b in range(NBUF):
        j = base + b
        pltpu.make_async_copy(
            tab_hbm.at[src_v.at[j]], bufs.at[b], gsem.at[b]).wait()
        pltpu.async_copy(bufs.at[b], acc.at[dst_v.at[j]], ssem.at[b],
                         add=True)
      for b in range(NBUF):
        j = base + b
        pltpu.make_async_copy(
            bufs.at[b], acc.at[dst_v.at[j]], ssem.at[b]).wait()

        @pl.when(t + 1 < outer)
        def _():
          pltpu.async_copy(
              tab_hbm.at[src_v.at[j + NBUF]], bufs.at[b], gsem.at[b])

    plsc.subcore_barrier()
    pltpu.sync_copy(acc.at[rows], out_hbm.at[c, rows])

  return edge_pass(table, src2d, dst2d, pad_src, pad_dst, zeros_np)


def _tc_mm1(x_pk, w1_blk, pblk):
  m, kdim = x_pk.shape
  grid = (m // pblk,)

  def body(x_ref, w1_ref, out_ref):
    out_ref[...] = jnp.dot(x_ref[...], w1_ref[...],
                           preferred_element_type=jnp.float32)

  return pl.pallas_call(
      body,
      grid=grid,
      compiler_params=pltpu.CompilerParams(
          dimension_semantics=("parallel",)),
      in_specs=[
          pl.BlockSpec((pblk, kdim), lambda i: (i, 0)),
          pl.BlockSpec((kdim, 128), lambda i: (0, 0)),
      ],
      out_specs=pl.BlockSpec((pblk, 128), lambda i: (i, 0)),
      out_shape=jax.ShapeDtypeStruct((m, 128), jnp.float32),
  )(x_pk, w1_blk)


def _tc_scale(xw_pk, cpart_pk, pblk):
  m = xw_pk.shape[0]
  grid = (m // pblk,)

  def body(xw_ref, cp_ref, dis_ref, xws_ref):
    deg = cp_ref[0] + cp_ref[1] + 1.0
    dis = lax.rsqrt(deg)
    dis_ref[...] = dis
    xws_ref[...] = xw_ref[...] * dis

  return pl.pallas_call(
      body,
      grid=grid,
      compiler_params=pltpu.CompilerParams(
          dimension_semantics=("parallel",)),
      in_specs=[
          pl.BlockSpec((pblk, 128), lambda i: (i, 0)),
          pl.BlockSpec((2, pblk, 128), lambda i: (0, i, 0)),
      ],
      out_specs=[
          pl.BlockSpec((pblk, 128), lambda i: (i, 0)),
          pl.BlockSpec((pblk, 128), lambda i: (i, 0)),
      ],
      out_shape=[
          jax.ShapeDtypeStruct((m, 128), jnp.float32),
          jax.ShapeDtypeStruct((m, 128), jnp.float32),
      ],
  )(xw_pk, cpart_pk)


def _tc_mid(s1_pk, xws1_pk, dis_pk, b1_row, w2_blk, pblk):
  m = xws1_pk.shape[0]
  grid = (m // pblk,)

  def body(s_ref, xws_ref, dis_ref, b1_ref, w2_ref, out_ref):
    dis = dis_ref[...]
    h = (s_ref[0] + s_ref[1] + xws_ref[...]) * dis + b1_ref[...]
    h = jnp.maximum(h, 0.0)
    xw2 = jnp.dot(h, w2_ref[...], preferred_element_type=jnp.float32)
    out_ref[...] = xw2 * dis

  return pl.pallas_call(
      body,
      grid=grid,
      compiler_params=pltpu.CompilerParams(
          dimension_semantics=("parallel",)),
      in_specs=[
          pl.BlockSpec((2, pblk, 128), lambda i: (0, i, 0)),
          pl.BlockSpec((pblk, 128), lambda i: (i, 0)),
          pl.BlockSpec((pblk, 128), lambda i: (i, 0)),
          pl.BlockSpec((1, 128), lambda i: (0, 0)),
          pl.BlockSpec((128, 128), lambda i: (0, 0)),
      ],
      out_specs=pl.BlockSpec((pblk, 128), lambda i: (i, 0)),
      out_shape=jax.ShapeDtypeStruct((m, 128), jnp.float32),
  )(s1_pk, xws1_pk, dis_pk, b1_row, w2_blk)


def _tc_fin(s2_pk, xws2_pk, dis_pk, b2_row, pblk):
  m = xws2_pk.shape[0]
  grid = (m // pblk,)

  def body(s_ref, xws_ref, dis_ref, b2_ref, out_ref):
    out_ref[...] = ((s_ref[0] + s_ref[1] + xws_ref[...]) * dis_ref[...]
                    + b2_ref[...])

  return pl.pallas_call(
      body,
      grid=grid,
      compiler_params=pltpu.CompilerParams(
          dimension_semantics=("parallel",)),
      in_specs=[
          pl.BlockSpec((2, pblk, 128), lambda i: (0, i, 0)),
          pl.BlockSpec((pblk, 128), lambda i: (i, 0)),
          pl.BlockSpec((pblk, 128), lambda i: (i, 0)),
          pl.BlockSpec((1, 128), lambda i: (0, 0)),
      ],
      out_specs=pl.BlockSpec((pblk, 128), lambda i: (i, 0)),
      out_shape=jax.ShapeDtypeStruct((m, 128), jnp.float32),
  )(s2_pk, xws2_pk, dis_pk, b2_row)


def kernel(x, edge_index, W1, b1, W2, b2):
  n, d_in = x.shape
  e = edge_index.shape[1]

  blk = 1280
  n_pad = -(-n // blk) * blk
  m_pk = n_pad // 8

  assert e % CHUNK == 0
  real_rows = e // CHUNK
  n_chunks = -(-real_rows // NW)
  n_chunks = -(-n_chunks // NBUF) * NBUF
  pad_rows = n_chunks - real_rows % n_chunks if real_rows % n_chunks else 0

  src2d = edge_index[0].reshape(real_rows, CHUNK)
  dst2d = edge_index[1].reshape(real_rows, CHUNK)
  pad_idx = np.arange(pad_rows * CHUNK, dtype=np.int32)
  pad_src = jnp.asarray((pad_idx % n).reshape(pad_rows, CHUNK))
  pad_dst = jnp.asarray(
      (n + pad_idx % (n_pad - n)).reshape(pad_rows, CHUNK).astype(np.int32))

  zeros_np = jnp.asarray(np.zeros((n_pad, L), np.float32))
  ones_chunk = jnp.asarray(np.ones((CHUNK, L), np.float32))
  x_pk = jnp.concatenate(
      [x, jnp.zeros((n_pad - n, d_in), x.dtype)]).reshape(m_pk, 8 * d_in)

  d_hid = W1.shape[1]
  mask1 = np.zeros((8 * d_in, 128), np.float32)
  mask2 = np.zeros((128, 128), np.float32)
  for i in range(8):
    mask1[i * d_in:(i + 1) * d_in, i * d_hid:(i + 1) * d_hid] = 1.0
    mask2[i * L:(i + 1) * L, i * L:(i + 1) * L] = 1.0
  w1_blk = jnp.tile(W1, (8, 8)) * jnp.asarray(mask1)
  w2_blk = jnp.tile(W2, (8, 8)) * jnp.asarray(mask2)
  b1_row = jnp.tile(b1, 8).reshape(1, 128)
  b2_row = jnp.tile(b2, 8).reshape(1, 128)

  xw_pk = _tc_mm1(x_pk, w1_blk, pblk=640)
  cpart = _sc_hist(dst2d, pad_dst, zeros_np, ones_chunk, n_pad, n_chunks,
                   real_rows)
  cpart_pk = cpart.reshape(NC, m_pk, 128)
  dis_pk, xws1_pk = _tc_scale(xw_pk, cpart_pk, pblk=640)

  s1 = _sc_edge_pass(xws1_pk.reshape(n_pad, L), src2d, dst2d, pad_src,
                     pad_dst, zeros_np, n_pad, n_chunks, real_rows)
  xws2_pk = _tc_mid(s1.reshape(NC, m_pk, 128), xws1_pk, dis_pk,
                    b1_row, w2_blk, pblk=640)

  s2 = _sc_edge_pass(xws2_pk.reshape(n_pad, L), src2d, dst2d, pad_src,
                     pad_dst, zeros_np, n_pad, n_chunks, real_rows)
  out_pk = _tc_fin(s2.reshape(NC, m_pk, 128), xws2_pk, dis_pk, b2_row,
                   pblk=640)
  return out_pk.reshape(n_pad, L)[:n]


# speedup vs baseline: 1.1368x; 1.1368x over previous
import functools

import jax
import jax.numpy as jnp
import numpy as np
from jax import lax
from jax.experimental import pallas as pl
from jax.experimental.pallas import tpu as pltpu
from jax.experimental.pallas import tpu_sc as plsc

_SC_PARAMS = pltpu.CompilerParams(use_tc_tiling_on_sc=False)

NC = 2
NS = 16
NW = NC * NS
L = 16
CHUNK = 128
NBUF = 8


def _load_idx(er_hbm, pad_hbm, row, wid, idx_v, n_chunks, real_rows):
  wb = real_rows // n_chunks
  rem = real_rows % n_chunks

  @pl.when(wid < wb)
  def _():
    pltpu.sync_copy(er_hbm.at[row, pl.ds(wid * n_chunks, n_chunks)], idx_v)

  if rem:
    @pl.when(wid == wb)
    def _():
      pltpu.sync_copy(er_hbm.at[row, pl.ds(wb * n_chunks, rem)],
                      idx_v.at[pl.ds(0, rem)])
      pltpu.sync_copy(pad_hbm.at[row],
                      idx_v.at[pl.ds(rem, n_chunks - rem)])


def _sc_hist(er, pads, zeros_np, ones_chunk, n_pad, n_chunks, real_rows):
  mesh = plsc.VectorSubcoreMesh(core_axis_name="c", subcore_axis_name="s")
  rpt = n_pad // NS

  @functools.partial(
      pl.kernel,
      out_type=jax.ShapeDtypeStruct((NC, n_pad, L), jnp.float32),
      mesh=mesh,
      compiler_params=_SC_PARAMS,
      scratch_types=[
          pltpu.VMEM((n_chunks, CHUNK), jnp.int32),
          pltpu.VMEM((CHUNK, L), jnp.float32),
          pltpu.VMEM_SHARED((n_pad, L), jnp.float32),
          pltpu.SemaphoreType.DMA,
      ],
  )
  def hist(er_hbm, pad_hbm, zeros_hbm, ones_hbm, out_hbm,
           dst_v, ones_v, acc, sem):
    c = lax.axis_index("c")
    s = lax.axis_index("s")
    wid = c * NS + s
    rows = pl.ds(s * rpt, rpt)
    _load_idx(er_hbm, pad_hbm, 1, wid, dst_v, n_chunks, real_rows)
    pltpu.sync_copy(ones_hbm, ones_v)
    pltpu.sync_copy(zeros_hbm.at[rows], acc.at[rows])
    plsc.subcore_barrier()

    @pl.loop(0, n_chunks)
    def _(j):
      pltpu.async_copy(ones_v, acc.at[dst_v.at[j]], sem, add=True)

    @pl.loop(0, n_chunks)
    def _(j):
      pltpu.make_async_copy(ones_v, acc.at[dst_v.at[j]], sem).wait()

    plsc.subcore_barrier()
    pltpu.sync_copy(acc.at[rows], out_hbm.at[c, rows])

  return hist(er, pads, zeros_np, ones_chunk)


def _sc_edge_pass(table, er, pads, zeros_np, n_pad, n_chunks, real_rows):
  mesh = plsc.VectorSubcoreMesh(core_axis_name="c", subcore_axis_name="s")
  rpt = n_pad // NS
  outer = n_chunks // NBUF

  @functools.partial(
      pl.kernel,
      out_type=jax.ShapeDtypeStruct((NC, n_pad, L), jnp.float32),
      mesh=mesh,
      compiler_params=_SC_PARAMS,
      scratch_types=[
          pltpu.VMEM((n_chunks, CHUNK), jnp.int32),
          pltpu.VMEM((n_chunks, CHUNK), jnp.int32),
          pltpu.VMEM((NBUF, CHUNK, L), jnp.float32),
          pltpu.VMEM_SHARED((n_pad, L), jnp.float32),
          pltpu.SemaphoreType.DMA((NBUF,)),
          pltpu.SemaphoreType.DMA((NBUF,)),
      ],
  )
  def edge_pass(tab_hbm, er_hbm, pad_hbm, zeros_hbm, out_hbm,
                src_v, dst_v, bufs, acc, gsem, ssem):
    c = lax.axis_index("c")
    s = lax.axis_index("s")
    wid = c * NS + s
    rows = pl.ds(s * rpt, rpt)
    _load_idx(er_hbm, pad_hbm, 0, wid, src_v, n_chunks, real_rows)
    _load_idx(er_hbm, pad_hbm, 1, wid, dst_v, n_chunks, real_rows)
    pltpu.sync_copy(zeros_hbm.at[rows], acc.at[rows])
    plsc.subcore_barrier()

    for b in range(NBUF):
      pltpu.async_copy(tab_hbm.at[src_v.at[b]], bufs.at[b], gsem.at[b])

    @pl.loop(0, outer)
    def _(t):
      base = t * NBUF
      for b in range(NBUF):
        j = base + b
        pltpu.make_async_copy(
            tab_hbm.at[src_v.at[j]], bufs.at[b], gsem.at[b]).wait()
        pltpu.async_copy(bufs.at[b], acc.at[dst_v.at[j]], ssem.at[b],
                         add=True)
      for b in range(NBUF):
        j = base + b
        pltpu.make_async_copy(
            bufs.at[b], acc.at[dst_v.at[j]], ssem.at[b]).wait()

        @pl.when(t + 1 < outer)
        def _():
          pltpu.async_copy(
              tab_hbm.at[src_v.at[j + NBUF]], bufs.at[b], gsem.at[b])

    plsc.subcore_barrier()
    pltpu.sync_copy(acc.at[rows], out_hbm.at[c, rows])

  return edge_pass(table, er, pads, zeros_np)


def _tc_mm1(x_pk, w1_blk, pblk):
  m, kdim = x_pk.shape
  grid = (m // pblk,)

  def body(x_ref, w1_ref, out_ref):
    out_ref[...] = jnp.dot(x_ref[...], w1_ref[...],
                           preferred_element_type=jnp.float32)

  return pl.pallas_call(
      body,
      grid=grid,
      compiler_params=pltpu.CompilerParams(
          dimension_semantics=("parallel",)),
      in_specs=[
          pl.BlockSpec((pblk, kdim), lambda i: (i, 0)),
          pl.BlockSpec((kdim, 128), lambda i: (0, 0)),
      ],
      out_specs=pl.BlockSpec((pblk, 128), lambda i: (i, 0)),
      out_shape=jax.ShapeDtypeStruct((m, 128), jnp.float32),
  )(x_pk, w1_blk)


def _tc_scale(xw_pk, cpart_pk, pblk):
  m = xw_pk.shape[0]
  grid = (m // pblk,)

  def body(xw_ref, cp_ref, dis_ref, xws_ref):
    deg = cp_ref[0] + cp_ref[1] + 1.0
    dis = lax.rsqrt(deg)
    dis_ref[...] = dis
    xws_ref[...] = xw_ref[...] * dis

  return pl.pallas_call(
      body,
      grid=grid,
      compiler_params=pltpu.CompilerParams(
          dimension_semantics=("parallel",)),
      in_specs=[
          pl.BlockSpec((pblk, 128), lambda i: (i, 0)),
          pl.BlockSpec((2, pblk, 128), lambda i: (0, i, 0)),
      ],
      out_specs=[
          pl.BlockSpec((pblk, 128), lambda i: (i, 0)),
          pl.BlockSpec((pblk, 128), lambda i: (i, 0)),
      ],
      out_shape=[
          jax.ShapeDtypeStruct((m, 128), jnp.float32),
          jax.ShapeDtypeStruct((m, 128), jnp.float32),
      ],
  )(xw_pk, cpart_pk)


def _tc_mid(s1_pk, xws1_pk, dis_pk, b1_row, w2_blk, pblk):
  m = xws1_pk.shape[0]
  grid = (m // pblk,)

  def body(s_ref, xws_ref, dis_ref, b1_ref, w2_ref, out_ref):
    dis = dis_ref[...]
    h = (s_ref[0] + s_ref[1] + xws_ref[...]) * dis + b1_ref[...]
    h = jnp.maximum(h, 0.0)
    xw2 = jnp.dot(h, w2_ref[...], preferred_element_type=jnp.float32)
    out_ref[...] = xw2 * dis

  return pl.pallas_call(
      body,
      grid=grid,
      compiler_params=pltpu.CompilerParams(
          dimension_semantics=("parallel",)),
      in_specs=[
          pl.BlockSpec((2, pblk, 128), lambda i: (0, i, 0)),
          pl.BlockSpec((pblk, 128), lambda i: (i, 0)),
          pl.BlockSpec((pblk, 128), lambda i: (i, 0)),
          pl.BlockSpec((1, 128), lambda i: (0, 0)),
          pl.BlockSpec((128, 128), lambda i: (0, 0)),
      ],
      out_specs=pl.BlockSpec((pblk, 128), lambda i: (i, 0)),
      out_shape=jax.ShapeDtypeStruct((m, 128), jnp.float32),
  )(s1_pk, xws1_pk, dis_pk, b1_row, w2_blk)


def _tc_fin(s2_pk, xws2_pk, dis_pk, b2_row, pblk):
  m = xws2_pk.shape[0]
  grid = (m // pblk,)

  def body(s_ref, xws_ref, dis_ref, b2_ref, out_ref):
    out_ref[...] = ((s_ref[0] + s_ref[1] + xws_ref[...]) * dis_ref[...]
                    + b2_ref[...])

  return pl.pallas_call(
      body,
      grid=grid,
      compiler_params=pltpu.CompilerParams(
          dimension_semantics=("parallel",)),
      in_specs=[
          pl.BlockSpec((2, pblk, 128), lambda i: (0, i, 0)),
          pl.BlockSpec((pblk, 128), lambda i: (i, 0)),
          pl.BlockSpec((pblk, 128), lambda i: (i, 0)),
          pl.BlockSpec((1, 128), lambda i: (0, 0)),
      ],
      out_specs=pl.BlockSpec((pblk, 128), lambda i: (i, 0)),
      out_shape=jax.ShapeDtypeStruct((m, 128), jnp.float32),
  )(s2_pk, xws2_pk, dis_pk, b2_row)


def kernel(x, edge_index, W1, b1, W2, b2):
  n, d_in = x.shape
  e = edge_index.shape[1]

  blk = 1280
  n_pad = -(-n // blk) * blk
  m_pk = n_pad // 8

  assert e % CHUNK == 0
  real_rows = e // CHUNK
  n_chunks = -(-real_rows // NW)
  n_chunks = -(-n_chunks // NBUF) * NBUF
  pad_rows = n_chunks - real_rows % n_chunks if real_rows % n_chunks else 0

  er = edge_index.reshape(2, real_rows, CHUNK)
  pad_idx = np.arange(pad_rows * CHUNK, dtype=np.int32)
  pads = jnp.asarray(
      np.stack([pad_idx % n, n + pad_idx % (n_pad - n)]).astype(
          np.int32).reshape(2, pad_rows, CHUNK))

  zeros_np = jnp.asarray(np.zeros((n_pad, L), np.float32))
  ones_chunk = jnp.asarray(np.ones((CHUNK, L), np.float32))
  x_pk = jnp.concatenate(
      [x, jnp.zeros((n_pad - n, d_in), x.dtype)]).reshape(m_pk, 8 * d_in)

  d_hid = W1.shape[1]
  mask1 = np.zeros((8 * d_in, 128), np.float32)
  mask2 = np.zeros((128, 128), np.float32)
  for i in range(8):
    mask1[i * d_in:(i + 1) * d_in, i * d_hid:(i + 1) * d_hid] = 1.0
    mask2[i * L:(i + 1) * L, i * L:(i + 1) * L] = 1.0
  w1_blk = jnp.tile(W1, (8, 8)) * jnp.asarray(mask1)
  w2_blk = jnp.tile(W2, (8, 8)) * jnp.asarray(mask2)
  b1_row = jnp.tile(b1, 8).reshape(1, 128)
  b2_row = jnp.tile(b2, 8).reshape(1, 128)

  xw_pk = _tc_mm1(x_pk, w1_blk, pblk=640)
  cpart = _sc_hist(er, pads, zeros_np, ones_chunk, n_pad, n_chunks,
                   real_rows)
  cpart_pk = cpart.reshape(NC, m_pk, 128)
  dis_pk, xws1_pk = _tc_scale(xw_pk, cpart_pk, pblk=640)

  s1 = _sc_edge_pass(xws1_pk.reshape(n_pad, L), er, pads, zeros_np,
                     n_pad, n_chunks, real_rows)
  xws2_pk = _tc_mid(s1.reshape(NC, m_pk, 128), xws1_pk, dis_pk,
                    b1_row, w2_blk, pblk=640)

  s2 = _sc_edge_pass(xws2_pk.reshape(n_pad, L), er, pads, zeros_np,
                     n_pad, n_chunks, real_rows)
  out_pk = _tc_fin(s2.reshape(NC, m_pk, 128), xws2_pk, dis_pk, b2_row,
                   pblk=640)
  return out_pk.reshape(n_pad, L)[:n]
